# Initial kernel scaffold; baseline (speedup 1.0000x reference)
import functools

import jax
import jax.numpy as jnp
from jax.experimental import pallas as pl
from jax.experimental.pallas import tpu as pltpu

H = 1024
F = 4096
E = 8
T = 2048
TF = 512
NF = F // TF


def _moe_body(x_ref, wr_ref, wg_ref, wu_ref, wd_ref, out_ref, w_out_ref):
    e = pl.program_id(0)
    f = pl.program_id(1)

    @pl.when((e == 0) & (f == 0))
    def _init():
        w = jax.nn.relu(
            jax.lax.dot_general(x_ref[...], wr_ref[...],
                                (((1,), (1,)), ((), ())),
                                preferred_element_type=jnp.float32))
        w_out_ref[...] = w
        out_ref[...] = jnp.zeros_like(out_ref)

    x = x_ref[...]
    g = jax.lax.dot_general(x, wg_ref[0], (((1,), (1,)), ((), ())),
                            preferred_element_type=jnp.float32)
    u = jax.lax.dot_general(x, wu_ref[0], (((1,), (1,)), ((), ())),
                            preferred_element_type=jnp.float32)
    a = g * jax.nn.sigmoid(g) * u
    part = jax.lax.dot_general(a, wd_ref[0], (((1,), (1,)), ((), ())),
                               preferred_element_type=jnp.float32)
    w_col = w_out_ref[:, e][:, None]
    out_ref[...] += part * w_col


def _moe(x, Wr, Wg, Wu, Wd):
    out, w = pl.pallas_call(
        _moe_body,
        grid=(E, NF),
        in_specs=[
            pl.BlockSpec((T, H), lambda e, f: (0, 0)),
            pl.BlockSpec((E, H), lambda e, f: (0, 0)),
            pl.BlockSpec((1, TF, H), lambda e, f: (e, f, 0)),
            pl.BlockSpec((1, TF, H), lambda e, f: (e, f, 0)),
            pl.BlockSpec((1, H, TF), lambda e, f: (e, 0, f)),
        ],
        out_specs=[
            pl.BlockSpec((T, H), lambda e, f: (0, 0)),
            pl.BlockSpec((T, E), lambda e, f: (0, 0)),
        ],
        out_shape=[
            jax.ShapeDtypeStruct((T, H), jnp.float32),
            jax.ShapeDtypeStruct((T, E), jnp.float32),
        ],
        compiler_params=pltpu.CompilerParams(
            dimension_semantics=("arbitrary", "arbitrary"),
            vmem_limit_bytes=120 * 1024 * 1024,
        ),
    )(x, Wr, Wg, Wu, Wd)
    return out, w


def kernel(hidden_states, Wr, Wg, Wu, Wd):
    orig_shape = hidden_states.shape
    x = hidden_states.reshape(-1, orig_shape[-1])
    out, w = _moe(x, Wr, Wg, Wu, Wd)
    return (out.reshape(orig_shape), w.reshape(orig_shape[:-1] + (E,)))


# speedup vs baseline: 1.4558x; 1.4558x over previous
import functools

import jax
import jax.numpy as jnp
from jax.experimental import pallas as pl
from jax.experimental.pallas import tpu as pltpu

H = 1024
F = 4096
E = 8
T = 2048
TF = 512
NF = F // TF


def _moe_body(x_ref, wr_ref, wg_ref, wu_ref, wd_ref, out_ref, w_out_ref):
    e = pl.program_id(0)
    f = pl.program_id(1)

    @pl.when((e == 0) & (f == 0))
    def _init():
        w = jax.nn.relu(
            jax.lax.dot_general(x_ref[...], wr_ref[...],
                                (((1,), (1,)), ((), ())),
                                preferred_element_type=jnp.float32))
        w_out_ref[...] = w
        out_ref[...] = jnp.zeros_like(out_ref)

    x = x_ref[...]
    g = jax.lax.dot_general(x, wg_ref[0], (((1,), (1,)), ((), ())),
                            preferred_element_type=jnp.float32)
    u = jax.lax.dot_general(x, wu_ref[0], (((1,), (1,)), ((), ())),
                            preferred_element_type=jnp.float32)
    a = g * jax.nn.sigmoid(g) * u
    part = jax.lax.dot_general(a, wd_ref[0], (((1,), (1,)), ((), ())),
                               preferred_element_type=jnp.float32)
    sel = (jax.lax.broadcasted_iota(jnp.int32, (1, E), 1) == e)
    w_col = jnp.sum(jnp.where(sel, w_out_ref[...], 0.0), axis=1, keepdims=True)
    out_ref[...] += part * w_col


def _moe(x, Wr, Wg, Wu, Wd):
    out, w = pl.pallas_call(
        _moe_body,
        grid=(E, NF),
        in_specs=[
            pl.BlockSpec((T, H), lambda e, f: (0, 0)),
            pl.BlockSpec((E, H), lambda e, f: (0, 0)),
            pl.BlockSpec((1, TF, H), lambda e, f: (e, f, 0)),
            pl.BlockSpec((1, TF, H), lambda e, f: (e, f, 0)),
            pl.BlockSpec((1, H, TF), lambda e, f: (e, 0, f)),
        ],
        out_specs=[
            pl.BlockSpec((T, H), lambda e, f: (0, 0)),
            pl.BlockSpec((T, E), lambda e, f: (0, 0)),
        ],
        out_shape=[
            jax.ShapeDtypeStruct((T, H), jnp.float32),
            jax.ShapeDtypeStruct((T, E), jnp.float32),
        ],
        compiler_params=pltpu.CompilerParams(
            dimension_semantics=("arbitrary", "arbitrary"),
            vmem_limit_bytes=120 * 1024 * 1024,
        ),
    )(x, Wr, Wg, Wu, Wd)
    return out, w


def kernel(hidden_states, Wr, Wg, Wu, Wd):
    orig_shape = hidden_states.shape
    x = hidden_states.reshape(-1, orig_shape[-1])
    out, w = _moe(x, Wr, Wg, Wu, Wd)
    return (out.reshape(orig_shape), w.reshape(orig_shape[:-1] + (E,)))
